# baseline (speedup 1.0000x reference)
import functools

import jax
import jax.numpy as jnp
from jax.experimental import pallas as pl


D = 512
P = 576
M = 100000
C = 1000
K = 50
H = 4
DH = D // H
ALPHA = 1.0

MB = 1024
MP = 100352


def _normalize(x, axis=-1):
    return x / (jnp.linalg.norm(x, axis=axis, keepdims=True) + 1e-12)


def _sim_body(p_ref, m_ref, o_ref):
    i = pl.program_id(0)
    s = jax.lax.dot_general(
        p_ref[...], m_ref[...], (((1,), (1,)), ((), ())),
        preferred_element_type=jnp.float32)
    col = i * MB + jax.lax.broadcasted_iota(jnp.int32, (1, MB), 1)
    o_ref[...] = jnp.where(col < M, s, -1e30)


def _sim_matmul(patches, mem_padded):
    return pl.pallas_call(
        _sim_body,
        grid=(MP // MB,),
        in_specs=[
            pl.BlockSpec((P, D), lambda i: (0, 0)),
            pl.BlockSpec((MB, D), lambda i: (i, 0)),
        ],
        out_specs=pl.BlockSpec((P, MB), lambda i: (0, i)),
        out_shape=jax.ShapeDtypeStruct((P, MP), jnp.float32),
    )(patches, mem_padded)


def kernel(test_global, test_patches, memory_nodes, textual_anchors,
           class_sums, class_counts, Wq, Wk, Wv, Wo, W1, b1, W2, b2):
    safe_counts = jnp.clip(class_counts, 1.0, None)[:, None]
    visual_protos = _normalize(class_sums / safe_counts)
    protos = _normalize(textual_anchors + ALPHA * visual_protos)
    sys1_logits = 100.0 * test_global @ protos.T

    mem_padded = jnp.pad(memory_nodes, ((0, MP - M), (0, 0)))
    sim = _sim_matmul(test_patches, mem_padded)

    _, topk_idx = jax.lax.top_k(sim, K)
    active = jnp.take(memory_nodes, topk_idx.reshape(-1), axis=0).reshape(P, K, D)

    q = (test_patches @ Wq).reshape(P, 1, H, DH)
    k = (active.reshape(P * K, D) @ Wk).reshape(P, K, H, DH)
    v = (active.reshape(P * K, D) @ Wv).reshape(P, K, H, DH)
    scores = jnp.sum(q * k, axis=-1) / jnp.sqrt(float(DH))
    attn = jax.nn.softmax(scores, axis=1)
    agg = jnp.sum(attn[..., None] * v, axis=1).reshape(P, D)
    updated = _normalize(test_patches + agg @ Wo)

    ev = jax.nn.relu(updated @ W1 + b1) @ W2 + b2
    w = jax.nn.softmax(ev, axis=0)
    global_feat = _normalize(jnp.sum(updated * w, axis=0, keepdims=True))
    final_logits = 100.0 * global_feat @ protos.T
    return (sys1_logits, final_logits)


# speedup vs baseline: 3.1654x; 3.1654x over previous
import functools

import jax
import jax.numpy as jnp
from jax.experimental import pallas as pl


D = 512
P = 576
M = 100000
C = 1000
K = 50
H = 4
DH = D // H
ALPHA = 1.0

MB = 1024
MP = 100352
CH = 128
NC = MP // CH


def _normalize(x, axis=-1):
    return x / (jnp.linalg.norm(x, axis=axis, keepdims=True) + 1e-12)


def _sim_body(p_ref, m_ref, o_ref, cm_ref):
    i = pl.program_id(0)
    s = jax.lax.dot_general(
        p_ref[...], m_ref[...], (((1,), (1,)), ((), ())),
        preferred_element_type=jnp.float32)
    col = i * MB + jax.lax.broadcasted_iota(jnp.int32, (1, MB), 1)
    s = jnp.where(col < M, s, -1e30)
    o_ref[...] = s
    cm_ref[0] = jnp.max(s.reshape(P, MB // CH, CH), axis=-1)


def _sim_matmul(patches, mem_padded):
    return pl.pallas_call(
        _sim_body,
        grid=(MP // MB,),
        in_specs=[
            pl.BlockSpec((P, D), lambda i: (0, 0)),
            pl.BlockSpec((MB, D), lambda i: (i, 0)),
        ],
        out_specs=[
            pl.BlockSpec((P, MB), lambda i: (0, i)),
            pl.BlockSpec((1, P, MB // CH), lambda i: (i, 0, 0)),
        ],
        out_shape=[
            jax.ShapeDtypeStruct((P, MP), jnp.float32),
            jax.ShapeDtypeStruct((MP // MB, P, MB // CH), jnp.float32),
        ],
    )(patches, mem_padded)


def kernel(test_global, test_patches, memory_nodes, textual_anchors,
           class_sums, class_counts, Wq, Wk, Wv, Wo, W1, b1, W2, b2):
    safe_counts = jnp.clip(class_counts, 1.0, None)[:, None]
    visual_protos = _normalize(class_sums / safe_counts)
    protos = _normalize(textual_anchors + ALPHA * visual_protos)
    sys1_logits = 100.0 * test_global @ protos.T

    mem_padded = jnp.pad(memory_nodes, ((0, MP - M), (0, 0)))
    sim, cm3 = _sim_matmul(test_patches, mem_padded)
    cm = cm3.transpose(1, 0, 2).reshape(P, NC)

    _, cidx = jax.lax.top_k(cm, K)
    chunks = jnp.take_along_axis(
        sim.reshape(P, NC, CH), cidx[:, :, None], axis=1)
    _, pos = jax.lax.top_k(chunks.reshape(P, K * CH), K)
    topk_idx = (jnp.take_along_axis(cidx, pos // CH, axis=1) * CH
                + pos % CH)

    active = jnp.take(memory_nodes, topk_idx.reshape(-1), axis=0).reshape(P, K, D)

    q = (test_patches @ Wq).reshape(P, 1, H, DH)
    k = (active.reshape(P * K, D) @ Wk).reshape(P, K, H, DH)
    v = (active.reshape(P * K, D) @ Wv).reshape(P, K, H, DH)
    scores = jnp.sum(q * k, axis=-1) / jnp.sqrt(float(DH))
    attn = jax.nn.softmax(scores, axis=1)
    agg = jnp.sum(attn[..., None] * v, axis=1).reshape(P, D)
    updated = _normalize(test_patches + agg @ Wo)

    ev = jax.nn.relu(updated @ W1 + b1) @ W2 + b2
    w = jax.nn.softmax(ev, axis=0)
    global_feat = _normalize(jnp.sum(updated * w, axis=0, keepdims=True))
    final_logits = 100.0 * global_feat @ protos.T
    return (sys1_logits, final_logits)


# speedup vs baseline: 6.5073x; 2.0558x over previous
import functools

import jax
import jax.numpy as jnp
from jax import lax
from jax.experimental import pallas as pl
from jax.experimental.pallas import tpu as pltpu
from jax.experimental.pallas import tpu_sc as plsc


D = 512
P = 576
M = 100000
C = 1000
K = 50
H = 4
DH = D // H
ALPHA = 1.0

MB = 1024
MP = 100352
CH = 128
NC = MP // CH


def _normalize(x, axis=-1):
    return x / (jnp.linalg.norm(x, axis=axis, keepdims=True) + 1e-12)


def _sim_body(p_ref, m_ref, o_ref, cm_ref):
    i = pl.program_id(0)
    s = jax.lax.dot_general(
        p_ref[...], m_ref[...], (((1,), (1,)), ((), ())),
        preferred_element_type=jnp.float32)
    col = i * MB + jax.lax.broadcasted_iota(jnp.int32, (1, MB), 1)
    s = jnp.where(col < M, s, -1e30)
    o_ref[...] = s
    cm_ref[0] = jnp.max(s.reshape(P, MB // CH, CH), axis=-1)


def _sim_matmul(patches, mem_padded):
    return pl.pallas_call(
        _sim_body,
        grid=(MP // MB,),
        in_specs=[
            pl.BlockSpec((P, D), lambda i: (0, 0)),
            pl.BlockSpec((MB, D), lambda i: (i, 0)),
        ],
        out_specs=[
            pl.BlockSpec((P, MB), lambda i: (0, i)),
            pl.BlockSpec((1, P, MB // CH), lambda i: (i, 0, 0)),
        ],
        out_shape=[
            jax.ShapeDtypeStruct((P, MP), jnp.float32),
            jax.ShapeDtypeStruct((MP // MB, P, MB // CH), jnp.float32),
        ],
    )(patches, mem_padded)


NW = 32
RPW = P // NW
NCH = 64
KP = 56
CAP = 2048
KLO = -1073741825
KHI = 1069547520


def _key16(v):
    b = lax.bitcast_convert_type(v, jnp.int32)
    return b ^ ((b >> 31) & jnp.int32(0x7FFFFFFF))


def _splat_sum16(s, iota):
    for step in (1, 2, 4, 8):
        s = s + jnp.take_along_axis(s, iota ^ step, axis=0)
    return s


def _popcnt16(m, iota):
    return _splat_sum16(jnp.where(m, 1, 0), iota)[0]


def _bcast16(vec, j):
    return jnp.take_along_axis(vec, jnp.full((16,), j, jnp.int32), axis=0)


def _minsplat16(s, iota):
    for step in (1, 2, 4, 8):
        s = jnp.minimum(s, jnp.take_along_axis(s, iota ^ step, axis=0))
    return s


def _append_lanes(pairs, m, off, iota, cap):
    cnt = _popcnt16(m, iota)

    def one(_, st):
        m_cur, o2 = st
        j = _minsplat16(jnp.where(m_cur, iota, 16), iota)[0]
        for ref, vec in pairs:
            ref[pl.ds(o2, 16)] = _bcast16(vec, j)
        return (m_cur & (iota != j), jnp.minimum(o2 + 1, cap))

    _, off = lax.fori_loop(0, cnt, one, (m, off))
    return off


def _sc_body(simtab, cm, mem, active,
             cmf, cmu, candc, gidx, chunkbuf, candu, candi, outi, midx,
             rows, sem):
    c = lax.axis_index("c")
    s = lax.axis_index("s")
    wid = s * 2 + c
    iota = lax.iota(jnp.int32, 16)
    zero16 = jnp.zeros((16,), jnp.int32)

    def row_body(i, _carry):
        r = wid * RPW + i
        pltpu.sync_copy(cm.at[r], cmf)

        def tr(t, _):
            cmu[pl.ds(t * 16, 16)] = _key16(cmf[pl.ds(t * 16, 16)])
            return 0
        lax.fori_loop(0, NC // 16, tr, 0)

        def bis(_, lohi):
            lo, hi = lohi
            mid = lo + (hi - lo) // 2

            def cnt(t, acc):
                return acc + jnp.where(cmu[pl.ds(t * 16, 16)] >= mid, 1, 0)
            cnum = _splat_sum16(lax.fori_loop(0, NC // 16, cnt, zero16),
                                iota)[0]
            take = cnum >= K
            return jnp.where(take, mid, lo), jnp.where(take, hi, mid)
        lo, _ = lax.fori_loop(0, 31, bis, (jnp.int32(KLO), jnp.int32(KHI)))

        for t in range(5):
            candc[pl.ds(t * 16, 16)] = jnp.full((16,), NC - 1, jnp.int32)

        def csel(t, off):
            m = cmu[pl.ds(t * 16, 16)] >= lo
            return _append_lanes([(candc, t * 16 + iota)], m, off, iota,
                                 jnp.int32(NCH))
        ncc = lax.fori_loop(0, NC // 16, csel, jnp.int32(0))

        base = r * NC
        for t in range(NCH // 16):
            gidx[pl.ds(t * 16, 16)] = base + candc[pl.ds(t * 16, 16)]
        pltpu.async_copy(simtab.at[gidx], chunkbuf, sem).wait()

        def fil(s2, off2):
            cvec = candc[pl.ds((s2 // 16) * 16, 16)]
            colbase = _bcast16(cvec, s2 % 16) * CH
            sv = jnp.full((16,), s2, jnp.int32)
            acc = off2
            for t2 in range(CH // 16):
                kk = _key16(chunkbuf[s2, pl.ds(t2 * 16, 16)])
                vm = (sv - ncc) >> 31
                kk = (kk & vm) | (jnp.int32(KLO) & ~vm)
                acc = _append_lanes(
                    [(candu, kk), (candi, colbase + t2 * 16 + iota)],
                    kk >= lo, acc, iota, jnp.int32(CAP))
            return acc
        off2 = lax.fori_loop(0, NCH, fil, jnp.int32(0))
        candu[pl.ds(off2, 16)] = jnp.full((16,), KLO, jnp.int32)
        nvd = (off2 + 15) // 16

        def bis2(_, lohi):
            lo2, hi2 = lohi
            mid = lo2 + (hi2 - lo2) // 2

            def cnt(t, acc):
                return acc + jnp.where(candu[pl.ds(t * 16, 16)] >= mid, 1, 0)
            cnum = _splat_sum16(lax.fori_loop(0, nvd, cnt, zero16), iota)[0]
            take = cnum >= K
            return jnp.where(take, mid, lo2), jnp.where(take, hi2, mid)
        tau, _ = lax.fori_loop(0, 31, bis2, (lo, jnp.int32(KHI)))

        for t in range(NCH // 16):
            outi[pl.ds(t * 16, 16)] = zero16

        def sel_above(t, off3):
            m = candu[pl.ds(t * 16, 16)] > tau
            return _append_lanes([(outi, candi[pl.ds(t * 16, 16)])], m, off3,
                                 iota, jnp.int32(CAP))
        off3 = lax.fori_loop(0, nvd, sel_above, jnp.int32(0))

        def sel_equal(t, off3):
            m = candu[pl.ds(t * 16, 16)] == tau
            return _append_lanes([(outi, candi[pl.ds(t * 16, 16)])], m, off3,
                                 iota, jnp.int32(CAP))
        lax.fori_loop(0, nvd, sel_equal, off3)

        for t in range(3):
            midx[pl.ds(t * 16, 16)] = outi[pl.ds(t * 16, 16)]
        midx[pl.ds(KP - 16, 16)] = outi[pl.ds(KP - 16, 16)]
        pltpu.async_copy(mem.at[midx], rows, sem).wait()
        pltpu.sync_copy(rows, active.at[r])
        return 0

    lax.fori_loop(0, RPW, row_body, 0)


def _sc_topk_gather(sim, cm, memory_nodes):
    simtab = sim.reshape(P * NC, CH)
    mesh = plsc.VectorSubcoreMesh(core_axis_name="c", subcore_axis_name="s")
    return pl.kernel(
        _sc_body,
        out_type=jax.ShapeDtypeStruct((P, KP, D), jnp.float32),
        mesh=mesh,
        scratch_types=[
            pltpu.VMEM((NC,), jnp.float32),
            pltpu.VMEM((NC,), jnp.int32),
            pltpu.VMEM((128,), jnp.int32),
            pltpu.VMEM((NCH,), jnp.int32),
            pltpu.VMEM((NCH, CH), jnp.float32),
            pltpu.VMEM((CAP + 128,), jnp.int32),
            pltpu.VMEM((CAP + 128,), jnp.int32),
            pltpu.VMEM((CAP + 128,), jnp.int32),
            pltpu.VMEM((KP,), jnp.int32),
            pltpu.VMEM((KP, D), jnp.float32),
            pltpu.SemaphoreType.DMA,
        ],
    )(simtab, cm, memory_nodes)


def kernel(test_global, test_patches, memory_nodes, textual_anchors,
           class_sums, class_counts, Wq, Wk, Wv, Wo, W1, b1, W2, b2):
    safe_counts = jnp.clip(class_counts, 1.0, None)[:, None]
    visual_protos = _normalize(class_sums / safe_counts)
    protos = _normalize(textual_anchors + ALPHA * visual_protos)
    sys1_logits = 100.0 * test_global @ protos.T

    mem_padded = jnp.pad(memory_nodes, ((0, MP - M), (0, 0)))
    sim, cm3 = _sim_matmul(test_patches, mem_padded)
    cm = cm3.transpose(1, 0, 2).reshape(P, NC)

    active = _sc_topk_gather(sim, cm, memory_nodes)[:, :K, :]

    q = (test_patches @ Wq).reshape(P, 1, H, DH)
    k = (active.reshape(P * K, D) @ Wk).reshape(P, K, H, DH)
    v = (active.reshape(P * K, D) @ Wv).reshape(P, K, H, DH)
    scores = jnp.sum(q * k, axis=-1) / jnp.sqrt(float(DH))
    attn = jax.nn.softmax(scores, axis=1)
    agg = jnp.sum(attn[..., None] * v, axis=1).reshape(P, D)
    updated = _normalize(test_patches + agg @ Wo)

    ev = jax.nn.relu(updated @ W1 + b1) @ W2 + b2
    w = jax.nn.softmax(ev, axis=0)
    global_feat = _normalize(jnp.sum(updated * w, axis=0, keepdims=True))
    final_logits = 100.0 * global_feat @ protos.T
    return (sys1_logits, final_logits)


# speedup vs baseline: 6.7562x; 1.0383x over previous
import functools

import jax
import jax.numpy as jnp
from jax import lax
from jax.experimental import pallas as pl
from jax.experimental.pallas import tpu as pltpu
from jax.experimental.pallas import tpu_sc as plsc


D = 512
P = 576
M = 100000
C = 1000
K = 50
H = 4
DH = D // H
ALPHA = 1.0

MB = 1024
MP = 100352
CH = 128
NC = MP // CH


def _normalize(x, axis=-1):
    return x / (jnp.linalg.norm(x, axis=axis, keepdims=True) + 1e-12)


def _sim_body(p_ref, m_ref, o_ref, cm_ref):
    i = pl.program_id(0)
    s = jax.lax.dot_general(
        p_ref[...], m_ref[...], (((1,), (1,)), ((), ())),
        preferred_element_type=jnp.float32)
    col = i * MB + jax.lax.broadcasted_iota(jnp.int32, (1, MB), 1)
    s = jnp.where(col < M, s, -1e30)
    o_ref[...] = s
    cm_ref[0] = jnp.max(s.reshape(P, MB // CH, CH), axis=-1)


def _sim_matmul(patches, mem_padded):
    return pl.pallas_call(
        _sim_body,
        grid=(MP // MB,),
        in_specs=[
            pl.BlockSpec((P, D), lambda i: (0, 0)),
            pl.BlockSpec((MB, D), lambda i: (i, 0)),
        ],
        out_specs=[
            pl.BlockSpec((P, MB), lambda i: (0, i)),
            pl.BlockSpec((1, P, MB // CH), lambda i: (i, 0, 0)),
        ],
        out_shape=[
            jax.ShapeDtypeStruct((P, MP), jnp.float32),
            jax.ShapeDtypeStruct((MP // MB, P, MB // CH), jnp.float32),
        ],
    )(patches, mem_padded)


NW = 32
RPW = P // NW
NCH = 64
KP = 56
CAP = 2048
KLO = -1073741825
KHI = 1069547520


def _key16(v):
    b = lax.bitcast_convert_type(v, jnp.int32)
    return b ^ ((b >> 31) & jnp.int32(0x7FFFFFFF))


def _splat_sum16(s, iota):
    for step in (1, 2, 4, 8):
        s = s + jnp.take_along_axis(s, iota ^ step, axis=0)
    return s


def _popcnt16(m, iota):
    return _splat_sum16(jnp.where(m, 1, 0), iota)[0]


def _bcast16(vec, j):
    return jnp.take_along_axis(vec, jnp.full((16,), j, jnp.int32), axis=0)


def _minsplat16(s, iota):
    for step in (1, 2, 4, 8):
        s = jnp.minimum(s, jnp.take_along_axis(s, iota ^ step, axis=0))
    return s


def _append_lanes(pairs, m, off, iota, cap):
    cnt = _popcnt16(m, iota)

    def one(_, st):
        m_cur, o2 = st
        j = _minsplat16(jnp.where(m_cur, iota, 16), iota)[0]
        for ref, vec in pairs:
            ref[pl.ds(o2, 16)] = _bcast16(vec, j)
        return (m_cur & (iota != j), jnp.minimum(o2 + 1, cap))

    _, off = lax.fori_loop(0, cnt, one, (m, off))
    return off


def _sc_body(simtab, cm, mem, active,
             cmf, cmu, candc, gidx, chunkbuf, candu, candi, outi, midx,
             rows, sem):
    c = lax.axis_index("c")
    s = lax.axis_index("s")
    wid = s * 2 + c
    iota = lax.iota(jnp.int32, 16)
    zero16 = jnp.zeros((16,), jnp.int32)

    def row_body(i, _carry):
        r = wid * RPW + i
        pltpu.sync_copy(cm.at[r], cmf)

        def tr(t, _):
            cmu[pl.ds(t * 16, 16)] = _key16(cmf[pl.ds(t * 16, 16)])
            return 0
        lax.fori_loop(0, NC // 16, tr, 0)

        def bis(_, lohi):
            lo, hi = lohi
            mid = lo + (hi - lo) // 2

            def cnt(t, acc):
                return acc + jnp.where(cmu[pl.ds(t * 16, 16)] >= mid, 1, 0)
            cnum = _splat_sum16(lax.fori_loop(0, NC // 16, cnt, zero16),
                                iota)[0]
            take = cnum >= K
            return jnp.where(take, mid, lo), jnp.where(take, hi, mid)
        lo, _ = lax.fori_loop(0, 31, bis, (jnp.int32(KLO), jnp.int32(KHI)))

        for t in range(5):
            candc[pl.ds(t * 16, 16)] = jnp.full((16,), NC - 1, jnp.int32)

        def csel(t, off):
            m = cmu[pl.ds(t * 16, 16)] >= lo
            return _append_lanes([(candc, t * 16 + iota)], m, off, iota,
                                 jnp.int32(NCH))
        ncc = lax.fori_loop(0, NC // 16, csel, jnp.int32(0))

        base = r * NC
        for t in range(NCH // 16):
            gidx[pl.ds(t * 16, 16)] = base + candc[pl.ds(t * 16, 16)]
        pltpu.async_copy(simtab.at[gidx], chunkbuf, sem).wait()

        def fil(s2, off2):
            cvec = candc[pl.ds((s2 // 16) * 16, 16)]
            colbase = _bcast16(cvec, s2 % 16) * CH
            sv = jnp.full((16,), s2, jnp.int32)
            acc = off2
            for t2 in range(CH // 16):
                kk = _key16(chunkbuf[s2, pl.ds(t2 * 16, 16)])
                vm = (sv - ncc) >> 31
                kk = (kk & vm) | (jnp.int32(KLO) & ~vm)
                acc = _append_lanes(
                    [(candu, kk), (candi, colbase + t2 * 16 + iota)],
                    kk >= lo, acc, iota, jnp.int32(CAP))
            return acc
        off2 = lax.fori_loop(0, NCH, fil, jnp.int32(0))
        candu[pl.ds(off2, 16)] = jnp.full((16,), KLO, jnp.int32)
        nvd = (off2 + 15) // 16

        def bis2(_, lohi):
            lo2, hi2 = lohi
            mid = lo2 + (hi2 - lo2) // 2

            def cnt(t, acc):
                return acc + jnp.where(candu[pl.ds(t * 16, 16)] >= mid, 1, 0)
            cnum = _splat_sum16(lax.fori_loop(0, nvd, cnt, zero16), iota)[0]
            take = cnum >= K
            return jnp.where(take, mid, lo2), jnp.where(take, hi2, mid)
        tau, _ = lax.fori_loop(0, 31, bis2, (lo, jnp.int32(KHI)))

        for t in range(NCH // 16):
            outi[pl.ds(t * 16, 16)] = zero16

        def sel_above(t, off3):
            m = candu[pl.ds(t * 16, 16)] > tau
            return _append_lanes([(outi, candi[pl.ds(t * 16, 16)])], m, off3,
                                 iota, jnp.int32(CAP))
        off3 = lax.fori_loop(0, nvd, sel_above, jnp.int32(0))

        def sel_equal(t, off3):
            m = candu[pl.ds(t * 16, 16)] == tau
            return _append_lanes([(outi, candi[pl.ds(t * 16, 16)])], m, off3,
                                 iota, jnp.int32(CAP))
        lax.fori_loop(0, nvd, sel_equal, off3)

        for t in range(3):
            midx[pl.ds(t * 16, 16)] = outi[pl.ds(t * 16, 16)]
        midx[pl.ds(KP - 16, 16)] = outi[pl.ds(KP - 16, 16)]
        pltpu.async_copy(mem.at[midx], rows, sem).wait()
        pltpu.sync_copy(rows, active.at[r])
        return 0

    lax.fori_loop(0, RPW, row_body, 0)


def _sc_topk_gather(sim, cm, memory_nodes):
    simtab = sim.reshape(P * NC, CH)
    mesh = plsc.VectorSubcoreMesh(core_axis_name="c", subcore_axis_name="s")
    return pl.kernel(
        _sc_body,
        out_type=jax.ShapeDtypeStruct((P, KP, D), jnp.float32),
        mesh=mesh,
        scratch_types=[
            pltpu.VMEM((NC,), jnp.float32),
            pltpu.VMEM((NC,), jnp.int32),
            pltpu.VMEM((128,), jnp.int32),
            pltpu.VMEM((NCH,), jnp.int32),
            pltpu.VMEM((NCH, CH), jnp.float32),
            pltpu.VMEM((CAP + 128,), jnp.int32),
            pltpu.VMEM((CAP + 128,), jnp.int32),
            pltpu.VMEM((CAP + 128,), jnp.int32),
            pltpu.VMEM((KP,), jnp.int32),
            pltpu.VMEM((KP, D), jnp.float32),
            pltpu.SemaphoreType.DMA,
        ],
    )(simtab, cm, memory_nodes)


PB = 64


def _attn_body(p_ref, a_ref, wq_ref, wk_ref, wv_ref, wo_ref, w1_ref, b1_ref,
               w2_ref, b2_ref, upd_ref, ev_ref):
    patches = p_ref[...]
    act = a_ref[...].reshape(PB * KP, D)
    q = jnp.dot(patches, wq_ref[...],
                preferred_element_type=jnp.float32)
    kp = jnp.dot(act, wk_ref[...], preferred_element_type=jnp.float32)
    vp = jnp.dot(act, wv_ref[...], preferred_element_type=jnp.float32)
    kp = kp.reshape(PB, KP, D)
    vp = vp.reshape(PB, KP, D)
    kmask = jax.lax.broadcasted_iota(jnp.int32, (PB, KP), 1) < K
    agg_parts = []
    for h in range(H):
        qh = q[:, h * DH:(h + 1) * DH]
        kh = kp[:, :, h * DH:(h + 1) * DH]
        vh = vp[:, :, h * DH:(h + 1) * DH]
        sc = jnp.sum(qh[:, None, :] * kh, axis=-1) / jnp.sqrt(float(DH))
        sc = jnp.where(kmask, sc, -1e30)
        sc = sc - jnp.max(sc, axis=1, keepdims=True)
        e = jnp.exp(sc)
        attn = e / jnp.sum(e, axis=1, keepdims=True)
        agg_parts.append(jnp.sum(attn[:, :, None] * vh, axis=1))
    agg = jnp.concatenate(agg_parts, axis=1)
    upd = patches + jnp.dot(agg, wo_ref[...],
                            preferred_element_type=jnp.float32)
    upd = upd / (jnp.sqrt(jnp.sum(upd * upd, axis=1, keepdims=True)) + 1e-12)
    upd_ref[...] = upd
    hid = jnp.maximum(
        jnp.dot(upd, w1_ref[...], preferred_element_type=jnp.float32)
        + b1_ref[...], 0.0)
    ev_ref[...] = (jnp.dot(hid, w2_ref[...],
                           preferred_element_type=jnp.float32) + b2_ref[...])


def _attention(test_patches, active, Wq, Wk, Wv, Wo, W1, b1, W2, b2):
    nb = P // PB
    return pl.pallas_call(
        _attn_body,
        grid=(nb,),
        in_specs=[
            pl.BlockSpec((PB, D), lambda i: (i, 0)),
            pl.BlockSpec((PB, KP, D), lambda i: (i, 0, 0)),
            pl.BlockSpec((D, D), lambda i: (0, 0)),
            pl.BlockSpec((D, D), lambda i: (0, 0)),
            pl.BlockSpec((D, D), lambda i: (0, 0)),
            pl.BlockSpec((D, D), lambda i: (0, 0)),
            pl.BlockSpec((D, D // 2), lambda i: (0, 0)),
            pl.BlockSpec((1, D // 2), lambda i: (0, 0)),
            pl.BlockSpec((D // 2, 1), lambda i: (0, 0)),
            pl.BlockSpec((1, 1), lambda i: (0, 0)),
        ],
        out_specs=[
            pl.BlockSpec((PB, D), lambda i: (i, 0)),
            pl.BlockSpec((PB, 1), lambda i: (i, 0)),
        ],
        out_shape=[
            jax.ShapeDtypeStruct((P, D), jnp.float32),
            jax.ShapeDtypeStruct((P, 1), jnp.float32),
        ],
    )(test_patches, active, Wq, Wk, Wv, Wo, W1, b1.reshape(1, D // 2),
      W2, b2.reshape(1, 1))


def _final_body(tg_ref, ta_ref, cs_ref, cc_ref, upd_ref, ev_ref,
                s1_ref, fl_ref):
    counts = jnp.maximum(cc_ref[...], 1.0)
    vp = cs_ref[...] / counts
    vp = vp / (jnp.sqrt(jnp.sum(vp * vp, axis=1, keepdims=True)) + 1e-12)
    protos = ta_ref[...] + ALPHA * vp
    protos = protos / (jnp.sqrt(jnp.sum(protos * protos, axis=1,
                                        keepdims=True)) + 1e-12)
    s1_ref[...] = 100.0 * jnp.dot(tg_ref[...], protos.T,
                                  preferred_element_type=jnp.float32)
    ev = ev_ref[...]
    ev = ev - jnp.max(ev, axis=1, keepdims=True)
    e = jnp.exp(ev)
    w = e / jnp.sum(e, axis=1, keepdims=True)
    gf = jnp.dot(w, upd_ref[...], preferred_element_type=jnp.float32)
    gf = gf / (jnp.sqrt(jnp.sum(gf * gf, axis=1, keepdims=True)) + 1e-12)
    fl_ref[...] = 100.0 * jnp.dot(gf, protos.T,
                                  preferred_element_type=jnp.float32)


def _final_stage(test_global, textual_anchors, class_sums, class_counts,
                 updated, ev):
    return pl.pallas_call(
        _final_body,
        out_shape=[
            jax.ShapeDtypeStruct((1, C), jnp.float32),
            jax.ShapeDtypeStruct((1, C), jnp.float32),
        ],
    )(test_global, textual_anchors, class_sums,
      class_counts.reshape(C, 1), updated, ev.reshape(1, P))


def kernel(test_global, test_patches, memory_nodes, textual_anchors,
           class_sums, class_counts, Wq, Wk, Wv, Wo, W1, b1, W2, b2):
    mem_padded = jnp.pad(memory_nodes, ((0, MP - M), (0, 0)))
    sim, cm3 = _sim_matmul(test_patches, mem_padded)
    cm = cm3.transpose(1, 0, 2).reshape(P, NC)

    active = _sc_topk_gather(sim, cm, memory_nodes)

    updated, ev = _attention(test_patches, active,
                             Wq, Wk, Wv, Wo, W1, b1, W2, b2)
    sys1_logits, final_logits = _final_stage(
        test_global, textual_anchors, class_sums, class_counts, updated, ev)
    return (sys1_logits, final_logits)


# speedup vs baseline: 7.1421x; 1.0571x over previous
import functools

import jax
import jax.numpy as jnp
from jax import lax
from jax.experimental import pallas as pl
from jax.experimental.pallas import tpu as pltpu
from jax.experimental.pallas import tpu_sc as plsc


D = 512
P = 576
M = 100000
C = 1000
K = 50
H = 4
DH = D // H
ALPHA = 1.0

MB = 1024
MP = 100352
CH = 128
NC = MP // CH


def _normalize(x, axis=-1):
    return x / (jnp.linalg.norm(x, axis=axis, keepdims=True) + 1e-12)


def _sim_body(p_ref, m_ref, o_ref, cm_ref):
    i = pl.program_id(0)
    s = jax.lax.dot_general(
        p_ref[...], m_ref[...], (((1,), (1,)), ((), ())),
        preferred_element_type=jnp.float32)
    col = i * MB + jax.lax.broadcasted_iota(jnp.int32, (1, MB), 1)
    s = jnp.where(col < M, s, -1e30)
    o_ref[...] = s
    cm_ref[0] = jnp.max(s.reshape(P, MB // CH, CH), axis=-1)


def _sim_matmul(patches, mem_padded):
    return pl.pallas_call(
        _sim_body,
        grid=(MP // MB,),
        in_specs=[
            pl.BlockSpec((P, D), lambda i: (0, 0)),
            pl.BlockSpec((MB, D), lambda i: (i, 0)),
        ],
        out_specs=[
            pl.BlockSpec((P, MB), lambda i: (0, i)),
            pl.BlockSpec((1, P, MB // CH), lambda i: (i, 0, 0)),
        ],
        out_shape=[
            jax.ShapeDtypeStruct((P, MP), jnp.float32),
            jax.ShapeDtypeStruct((MP // MB, P, MB // CH), jnp.float32),
        ],
    )(patches, mem_padded)


NW = 32
RPW = P // NW
NCH = 64
KP = 56
CAP = 2048
KLO = -1073741825
KHI = 1069547520


def _key16(v):
    b = lax.bitcast_convert_type(v, jnp.int32)
    return b ^ ((b >> 31) & jnp.int32(0x7FFFFFFF))


def _splat_sum16(s, iota):
    for step in (1, 2, 4, 8):
        s = s + jnp.take_along_axis(s, iota ^ step, axis=0)
    return s


def _popcnt16(m, iota):
    return _splat_sum16(jnp.where(m, 1, 0), iota)[0]


def _bcast16(vec, j):
    return jnp.take_along_axis(vec, jnp.full((16,), j, jnp.int32), axis=0)


def _minsplat16(s, iota):
    for step in (1, 2, 4, 8):
        s = jnp.minimum(s, jnp.take_along_axis(s, iota ^ step, axis=0))
    return s


def _append_lanes(pairs, m, off, iota, cap):
    cnt = _popcnt16(m, iota)

    def one(_, st):
        m_cur, o2 = st
        j = _minsplat16(jnp.where(m_cur, iota, 16), iota)[0]
        for ref, vec in pairs:
            ref[pl.ds(o2, 16)] = _bcast16(vec, j)
        return (m_cur & (iota != j), jnp.minimum(o2 + 1, cap))

    _, off = lax.fori_loop(0, cnt, one, (m, off))
    return off


def _sc_body(simtab, cm, mem, active,
             cmf, cmu, candc, gidx, chunkbuf, candu, candi, outi, midx,
             rows, sem):
    c = lax.axis_index("c")
    s = lax.axis_index("s")
    wid = s * 2 + c
    iota = lax.iota(jnp.int32, 16)
    zero16 = jnp.zeros((16,), jnp.int32)

    def row_body(i, _carry):
        r = wid * RPW + i
        pltpu.sync_copy(cm.at[r], cmf)

        def tr(t, _):
            cmu[pl.ds(t * 16, 16)] = _key16(cmf[pl.ds(t * 16, 16)])
            return 0
        lax.fori_loop(0, NC // 16, tr, 0)

        def bis(_, lohi):
            lo, hi = lohi
            mid = lo + (hi - lo) // 2

            def cnt(t, acc):
                return acc + jnp.where(cmu[pl.ds(t * 16, 16)] >= mid, 1, 0)
            cnum = _splat_sum16(lax.fori_loop(0, NC // 16, cnt, zero16),
                                iota)[0]
            take = cnum >= K
            return jnp.where(take, mid, lo), jnp.where(take, hi, mid)
        lo, _ = lax.fori_loop(0, 31, bis, (jnp.int32(KLO), jnp.int32(KHI)))

        for t in range(5):
            candc[pl.ds(t * 16, 16)] = jnp.full((16,), NC - 1, jnp.int32)

        def csel(t, off):
            m = cmu[pl.ds(t * 16, 16)] >= lo
            return _append_lanes([(candc, t * 16 + iota)], m, off, iota,
                                 jnp.int32(NCH))
        ncc = lax.fori_loop(0, NC // 16, csel, jnp.int32(0))

        base = r * NC
        for t in range(NCH // 16):
            gidx[pl.ds(t * 16, 16)] = base + candc[pl.ds(t * 16, 16)]
        pltpu.async_copy(simtab.at[gidx], chunkbuf, sem).wait()

        def fil(s2, off2):
            cvec = candc[pl.ds((s2 // 16) * 16, 16)]
            colbase = _bcast16(cvec, s2 % 16) * CH
            acc = off2
            for t2 in range(CH // 16):
                kk = _key16(chunkbuf[s2, pl.ds(t2 * 16, 16)])
                acc = _append_lanes(
                    [(candu, kk), (candi, colbase + t2 * 16 + iota)],
                    kk >= lo, acc, iota, jnp.int32(CAP))
            return acc
        off2 = lax.fori_loop(0, jnp.minimum(ncc, jnp.int32(NCH)), fil,
                             jnp.int32(0))
        candu[pl.ds(off2, 16)] = jnp.full((16,), KLO, jnp.int32)
        nvd = (off2 + 15) // 16

        def bis2(_, lohi):
            lo2, hi2 = lohi
            mid = lo2 + (hi2 - lo2) // 2

            def cnt(t, acc):
                return acc + jnp.where(candu[pl.ds(t * 16, 16)] >= mid, 1, 0)
            cnum = _splat_sum16(lax.fori_loop(0, nvd, cnt, zero16), iota)[0]
            take = cnum >= K
            return jnp.where(take, mid, lo2), jnp.where(take, hi2, mid)
        tau, _ = lax.fori_loop(0, 31, bis2, (lo, jnp.int32(KHI)))

        for t in range(NCH // 16):
            outi[pl.ds(t * 16, 16)] = zero16

        def sel_above(t, off3):
            m = candu[pl.ds(t * 16, 16)] > tau
            return _append_lanes([(outi, candi[pl.ds(t * 16, 16)])], m, off3,
                                 iota, jnp.int32(CAP))
        off3 = lax.fori_loop(0, nvd, sel_above, jnp.int32(0))

        def sel_equal(t, off3):
            m = candu[pl.ds(t * 16, 16)] == tau
            return _append_lanes([(outi, candi[pl.ds(t * 16, 16)])], m, off3,
                                 iota, jnp.int32(CAP))
        lax.fori_loop(0, nvd, sel_equal, off3)

        for t in range(3):
            midx[pl.ds(t * 16, 16)] = outi[pl.ds(t * 16, 16)]
        midx[pl.ds(KP - 16, 16)] = outi[pl.ds(KP - 16, 16)]
        pltpu.async_copy(mem.at[midx], rows, sem).wait()
        pltpu.sync_copy(rows, active.at[r])
        return 0

    lax.fori_loop(0, RPW, row_body, 0)


def _sc_topk_gather(sim, cm, memory_nodes):
    simtab = sim.reshape(P * NC, CH)
    mesh = plsc.VectorSubcoreMesh(core_axis_name="c", subcore_axis_name="s")
    return pl.kernel(
        _sc_body,
        out_type=jax.ShapeDtypeStruct((P, KP, D), jnp.float32),
        mesh=mesh,
        scratch_types=[
            pltpu.VMEM((NC,), jnp.float32),
            pltpu.VMEM((NC,), jnp.int32),
            pltpu.VMEM((128,), jnp.int32),
            pltpu.VMEM((NCH,), jnp.int32),
            pltpu.VMEM((NCH, CH), jnp.float32),
            pltpu.VMEM((CAP + 128,), jnp.int32),
            pltpu.VMEM((CAP + 128,), jnp.int32),
            pltpu.VMEM((CAP + 128,), jnp.int32),
            pltpu.VMEM((KP,), jnp.int32),
            pltpu.VMEM((KP, D), jnp.float32),
            pltpu.SemaphoreType.DMA,
        ],
    )(simtab, cm, memory_nodes)


PB = 64


def _attn_body(p_ref, a_ref, wq_ref, wk_ref, wv_ref, wo_ref, w1_ref, b1_ref,
               w2_ref, b2_ref, upd_ref, ev_ref):
    patches = p_ref[...]
    act = a_ref[...].reshape(PB * KP, D)
    q = jnp.dot(patches, wq_ref[...],
                preferred_element_type=jnp.float32)
    kp = jnp.dot(act, wk_ref[...], preferred_element_type=jnp.float32)
    vp = jnp.dot(act, wv_ref[...], preferred_element_type=jnp.float32)
    kp = kp.reshape(PB, KP, D)
    vp = vp.reshape(PB, KP, D)
    kmask = jax.lax.broadcasted_iota(jnp.int32, (PB, KP), 1) < K
    agg_parts = []
    for h in range(H):
        qh = q[:, h * DH:(h + 1) * DH]
        kh = kp[:, :, h * DH:(h + 1) * DH]
        vh = vp[:, :, h * DH:(h + 1) * DH]
        sc = jnp.sum(qh[:, None, :] * kh, axis=-1) / jnp.sqrt(float(DH))
        sc = jnp.where(kmask, sc, -1e30)
        sc = sc - jnp.max(sc, axis=1, keepdims=True)
        e = jnp.exp(sc)
        attn = e / jnp.sum(e, axis=1, keepdims=True)
        agg_parts.append(jnp.sum(attn[:, :, None] * vh, axis=1))
    agg = jnp.concatenate(agg_parts, axis=1)
    upd = patches + jnp.dot(agg, wo_ref[...],
                            preferred_element_type=jnp.float32)
    upd = upd / (jnp.sqrt(jnp.sum(upd * upd, axis=1, keepdims=True)) + 1e-12)
    upd_ref[...] = upd
    hid = jnp.maximum(
        jnp.dot(upd, w1_ref[...], preferred_element_type=jnp.float32)
        + b1_ref[...], 0.0)
    ev_ref[...] = (jnp.dot(hid, w2_ref[...],
                           preferred_element_type=jnp.float32) + b2_ref[...])


def _attention(test_patches, active, Wq, Wk, Wv, Wo, W1, b1, W2, b2):
    nb = P // PB
    return pl.pallas_call(
        _attn_body,
        grid=(nb,),
        in_specs=[
            pl.BlockSpec((PB, D), lambda i: (i, 0)),
            pl.BlockSpec((PB, KP, D), lambda i: (i, 0, 0)),
            pl.BlockSpec((D, D), lambda i: (0, 0)),
            pl.BlockSpec((D, D), lambda i: (0, 0)),
            pl.BlockSpec((D, D), lambda i: (0, 0)),
            pl.BlockSpec((D, D), lambda i: (0, 0)),
            pl.BlockSpec((D, D // 2), lambda i: (0, 0)),
            pl.BlockSpec((1, D // 2), lambda i: (0, 0)),
            pl.BlockSpec((D // 2, 1), lambda i: (0, 0)),
            pl.BlockSpec((1, 1), lambda i: (0, 0)),
        ],
        out_specs=[
            pl.BlockSpec((PB, D), lambda i: (i, 0)),
            pl.BlockSpec((PB, 1), lambda i: (i, 0)),
        ],
        out_shape=[
            jax.ShapeDtypeStruct((P, D), jnp.float32),
            jax.ShapeDtypeStruct((P, 1), jnp.float32),
        ],
    )(test_patches, active, Wq, Wk, Wv, Wo, W1, b1.reshape(1, D // 2),
      W2, b2.reshape(1, 1))


def _final_body(tg_ref, ta_ref, cs_ref, cc_ref, upd_ref, ev_ref,
                s1_ref, fl_ref):
    counts = jnp.maximum(cc_ref[...], 1.0)
    vp = cs_ref[...] / counts
    vp = vp / (jnp.sqrt(jnp.sum(vp * vp, axis=1, keepdims=True)) + 1e-12)
    protos = ta_ref[...] + ALPHA * vp
    protos = protos / (jnp.sqrt(jnp.sum(protos * protos, axis=1,
                                        keepdims=True)) + 1e-12)
    s1_ref[...] = 100.0 * jnp.dot(tg_ref[...], protos.T,
                                  preferred_element_type=jnp.float32)
    ev = ev_ref[...]
    ev = ev - jnp.max(ev, axis=1, keepdims=True)
    e = jnp.exp(ev)
    w = e / jnp.sum(e, axis=1, keepdims=True)
    gf = jnp.dot(w, upd_ref[...], preferred_element_type=jnp.float32)
    gf = gf / (jnp.sqrt(jnp.sum(gf * gf, axis=1, keepdims=True)) + 1e-12)
    fl_ref[...] = 100.0 * jnp.dot(gf, protos.T,
                                  preferred_element_type=jnp.float32)


def _final_stage(test_global, textual_anchors, class_sums, class_counts,
                 updated, ev):
    return pl.pallas_call(
        _final_body,
        out_shape=[
            jax.ShapeDtypeStruct((1, C), jnp.float32),
            jax.ShapeDtypeStruct((1, C), jnp.float32),
        ],
    )(test_global, textual_anchors, class_sums,
      class_counts.reshape(C, 1), updated, ev.reshape(1, P))


def kernel(test_global, test_patches, memory_nodes, textual_anchors,
           class_sums, class_counts, Wq, Wk, Wv, Wo, W1, b1, W2, b2):
    mem_padded = jnp.pad(memory_nodes, ((0, MP - M), (0, 0)))
    sim, cm3 = _sim_matmul(test_patches, mem_padded)
    cm = cm3.transpose(1, 0, 2).reshape(P, NC)

    active = _sc_topk_gather(sim, cm, memory_nodes)

    updated, ev = _attention(test_patches, active,
                             Wq, Wk, Wv, Wo, W1, b1, W2, b2)
    sys1_logits, final_logits = _final_stage(
        test_global, textual_anchors, class_sums, class_counts, updated, ev)
    return (sys1_logits, final_logits)
